# baseline (speedup 1.0000x reference)
import jax
import jax.numpy as jnp
from jax import lax
from jax.experimental import pallas as pl
from jax.experimental.pallas import tpu as pltpu
from jax.experimental.pallas import tpu_sc as plsc

_BATCH = 16
_N_NODES = 512
_HIDDEN = 768
_HALF = _N_NODES // 2


def _sc_body(node_hbm, tok_hbm, out_hbm, sem_big, sem_tok):
    c = lax.axis_index("c")
    s = lax.axis_index("s")
    wid = s * 2 + c
    b = wid // 2
    half = wid % 2
    r0 = half * _HALF

    big = pltpu.make_async_copy(
        node_hbm.at[b, pl.ds(r0, _HALF), :],
        out_hbm.at[b, pl.ds(r0 + 1, _HALF), :],
        sem_big,
    )
    big.start()

    @pl.when(half == 0)
    def _():
        tok = pltpu.make_async_copy(
            tok_hbm,
            out_hbm.at[b, pl.ds(0, 1), :],
            sem_tok,
        )
        tok.start()
        tok.wait()

    big.wait()


@jax.jit
def _sc_call(node_feature, graph_token):
    run = pl.kernel(
        _sc_body,
        out_type=jax.ShapeDtypeStruct((_BATCH, _N_NODES + 1, _HIDDEN), jnp.float32),
        mesh=plsc.VectorSubcoreMesh(core_axis_name="c", subcore_axis_name="s"),
        scratch_types=[pltpu.SemaphoreType.DMA, pltpu.SemaphoreType.DMA],
        compiler_params=pltpu.CompilerParams(use_tc_tiling_on_sc=False),
    )
    return run(node_feature, graph_token)


def kernel(node_feature, graph_token):
    return _sc_call(node_feature, graph_token)


# speedup vs baseline: 7.5741x; 7.5741x over previous
import jax
import jax.numpy as jnp
from jax import lax
from jax.experimental import pallas as pl
from jax.experimental.pallas import tpu as pltpu
from jax.experimental.pallas import tpu_sc as plsc

_BATCH = 16
_N_NODES = 512
_HIDDEN = 768
_HALF = _N_NODES // 2
_CH = 64
_NCH = _HALF // _CH


def _sc_body(node_hbm, tok_hbm, out_hbm, buf0, buf1, tokbuf,
             sem_in, sem_out, sem_tok):
    c = lax.axis_index("c")
    s = lax.axis_index("s")
    wid = s * 2 + c
    b = wid // 2
    half = wid % 2
    r0 = half * _HALF

    @pl.when(half == 0)
    def _():
        pltpu.sync_copy(tok_hbm, tokbuf)
        tok = pltpu.make_async_copy(tokbuf, out_hbm.at[b, pl.ds(0, 1), :],
                                    sem_tok)
        tok.start()
        tok.wait()

    bufs = (buf0, buf1)
    ins = [
        pltpu.make_async_copy(
            node_hbm.at[b, pl.ds(r0 + i * _CH, _CH), :],
            bufs[i % 2],
            sem_in.at[i],
        )
        for i in range(_NCH)
    ]
    outs = [
        pltpu.make_async_copy(
            bufs[i % 2],
            out_hbm.at[b, pl.ds(r0 + 1 + i * _CH, _CH), :],
            sem_out.at[i],
        )
        for i in range(_NCH)
    ]

    ins[0].start()
    ins[1].start()
    for i in range(_NCH):
        ins[i].wait()
        outs[i].start()
        if i + 2 < _NCH:
            outs[i].wait()
            ins[i + 2].start()
    outs[_NCH - 2].wait()
    outs[_NCH - 1].wait()


@jax.jit
def _sc_call(node_feature, graph_token):
    run = pl.kernel(
        _sc_body,
        out_type=jax.ShapeDtypeStruct((_BATCH, _N_NODES + 1, _HIDDEN),
                                      jnp.float32),
        mesh=plsc.VectorSubcoreMesh(core_axis_name="c", subcore_axis_name="s"),
        scratch_types=[
            pltpu.VMEM((_CH, _HIDDEN), jnp.float32),
            pltpu.VMEM((_CH, _HIDDEN), jnp.float32),
            pltpu.VMEM((1, _HIDDEN), jnp.float32),
            pltpu.SemaphoreType.DMA((_NCH,)),
            pltpu.SemaphoreType.DMA((_NCH,)),
            pltpu.SemaphoreType.DMA,
        ],
        compiler_params=pltpu.CompilerParams(use_tc_tiling_on_sc=False),
    )
    return run(node_feature, graph_token)


def kernel(node_feature, graph_token):
    return _sc_call(node_feature, graph_token)


# speedup vs baseline: 20.2525x; 2.6739x over previous
import jax
import jax.numpy as jnp
from jax.experimental import pallas as pl
from jax.experimental.pallas import tpu as pltpu

_BATCH = 16
_N_NODES = 512
_HIDDEN = 768


def _tc_body(node_ref, tok_ref, out_ref):
    out_ref[0, 0:1, :] = tok_ref[...]
    out_ref[0, 1:_N_NODES + 1, :] = node_ref[0]


@jax.jit
def _tc_call(node_feature, graph_token):
    return pl.pallas_call(
        _tc_body,
        grid=(_BATCH,),
        in_specs=[
            pl.BlockSpec((1, _N_NODES, _HIDDEN), lambda b: (b, 0, 0)),
            pl.BlockSpec((1, _HIDDEN), lambda b: (0, 0)),
        ],
        out_specs=pl.BlockSpec((1, _N_NODES + 1, _HIDDEN),
                               lambda b: (b, 0, 0)),
        out_shape=jax.ShapeDtypeStruct((_BATCH, _N_NODES + 1, _HIDDEN),
                                       jnp.float32),
    )(node_feature, graph_token)


def kernel(node_feature, graph_token):
    return _tc_call(node_feature, graph_token)
